# Initial kernel scaffold; baseline (speedup 1.0000x reference)
import jax
import jax.numpy as jnp
from jax.experimental import pallas as pl


def kernel(encoder_output, pitch_target, energy_target, pitch_table, energy_table):
    raise NotImplementedError("write your pallas kernel here")


# speedup vs baseline: 3.5927x; 3.5927x over previous
import functools

import jax
import jax.numpy as jnp
from jax import lax
from jax.experimental import pallas as pl
from jax.experimental.pallas import tpu as pltpu
from jax.experimental.pallas import tpu_sc as plsc

NC, NS, L = 2, 16, 16
NW = NC * NS
T = 64


def _sc_call(N, H, NBINS):
    TPW = N // NW
    CHUNKS = TPW // T
    CH = H // L

    mesh = plsc.VectorSubcoreMesh(core_axis_name="c", subcore_axis_name="s")

    @functools.partial(
        pl.kernel,
        out_type=jax.ShapeDtypeStruct((N, H), jnp.float32),
        mesh=mesh,
        compiler_params=pltpu.CompilerParams(needs_layout_passes=False),
        scratch_types=[
            pltpu.VMEM((T, H), jnp.float32),
            pltpu.VMEM((T, H), jnp.float32),
            pltpu.VMEM((T, H), jnp.float32),
            pltpu.VMEM((T,), jnp.float32),
            pltpu.VMEM((T,), jnp.float32),
            pltpu.VMEM((T,), jnp.int32),
            pltpu.VMEM((T,), jnp.int32),
            pltpu.VMEM((2 * NBINS,), jnp.float32),
            pltpu.SemaphoreType.DMA,
            pltpu.SemaphoreType.DMA,
        ],
    )
    def body(enc_hbm, pv_hbm, ev_hbm, ctab_hbm, bnd_hbm, out_hbm,
             out_buf, prows, erows, pvals, evals, pidx, eidx, bnds, sem_p, sem_e):
        wid = lax.axis_index("s") * NC + lax.axis_index("c")
        pltpu.sync_copy(bnd_hbm, bnds)

        def searchsorted(vals_ref, idx_ref, base_bin):
            for j in range(T // L):
                sl = pl.ds(j * L, L)
                v = vals_ref[sl]
                curr = jnp.zeros((L,), jnp.int32)
                step = NBINS // 2
                while step >= 1:
                    probe = plsc.load_gather(bnds, [curr + (base_bin + step - 1)])
                    curr = jnp.where(probe < v, curr + step, curr)
                    step //= 2
                idx_ref[sl] = curr + base_bin

        @pl.loop(0, CHUNKS)
        def _chunk(c):
            base = wid * TPW + c * T
            pltpu.sync_copy(pv_hbm.at[pl.ds(base, T)], pvals)
            pltpu.sync_copy(ev_hbm.at[pl.ds(base, T)], evals)
            searchsorted(pvals, pidx, 0)
            searchsorted(evals, eidx, NBINS)
            cp_p = pltpu.async_copy(ctab_hbm.at[pidx], prows, sem_p)
            cp_e = pltpu.async_copy(ctab_hbm.at[eidx], erows, sem_e)
            pltpu.sync_copy(enc_hbm.at[pl.ds(base, T)], out_buf)
            cp_p.wait()
            cp_e.wait()

            @pl.loop(0, T)
            def _row(t):
                for h in range(CH):
                    sl = pl.ds(h * L, L)
                    out_buf[t, sl] = out_buf[t, sl] + prows[t, sl] + erows[t, sl]

            pltpu.sync_copy(out_buf, out_hbm.at[pl.ds(base, T)])

    return body


def kernel(encoder_output, pitch_target, energy_target, pitch_table, energy_table):
    B, S, H = encoder_output.shape
    N = B * S
    NBINS = pitch_table.shape[0]
    enc = encoder_output.reshape(N, H)
    pv = pitch_target.reshape(N)
    ev = energy_target.reshape(N)
    ctab = jnp.concatenate([pitch_table, energy_table], axis=0)
    bnds = jnp.concatenate([
        jnp.linspace(50.0, 400.0, NBINS),
        jnp.linspace(0.0, 1.0, NBINS),
    ])
    out = _sc_call(N, H, NBINS)(enc, pv, ev, ctab, bnds)
    return out.reshape(B, S, H)
